# baseline (speedup 1.0000x reference)
import functools

import jax
import jax.numpy as jnp
from jax import lax
from jax.experimental import pallas as pl
from jax.experimental.pallas import tpu as pltpu
from jax.experimental.pallas import tpu_sc as plsc


@functools.partial(jax.jit, static_argnums=(2, 3, 4))
def _gather_call(input, weight, B, V, D):
    info = plsc.get_sparse_core_info()
    NW = info.num_cores * info.num_subcores
    t_per_w = B // NW
    mesh = plsc.VectorSubcoreMesh(core_axis_name="c", subcore_axis_name="s")

    @functools.partial(
        pl.kernel,
        mesh=mesh,
        compiler_params=pltpu.CompilerParams(use_tc_tiling_on_sc=False),
        out_type=jax.ShapeDtypeStruct((B, D), jnp.float32),
        scratch_types=[
            pltpu.VMEM((t_per_w,), jnp.int32),
            pltpu.VMEM((t_per_w, D), jnp.float32),
            pltpu.SemaphoreType.DMA,
        ],
    )
    def k(idx_hbm, table_hbm, out_hbm, idx_v, rows_v, sem):
        wid = lax.axis_index("s") * info.num_cores + lax.axis_index("c")
        base = wid * t_per_w
        pltpu.sync_copy(idx_hbm.at[pl.ds(base, t_per_w)], idx_v)
        pltpu.async_copy(table_hbm.at[idx_v], rows_v, sem).wait()
        pltpu.sync_copy(rows_v, out_hbm.at[pl.ds(base, t_per_w)])

    return k(input, weight)


def kernel(input, weight):
    (B,) = input.shape
    V, D = weight.shape
    return _gather_call(input, weight, B, V, D)


# speedup vs baseline: 2.0147x; 2.0147x over previous
import functools

import jax
import jax.numpy as jnp
from jax import lax
from jax.experimental import pallas as pl
from jax.experimental.pallas import tpu as pltpu
from jax.experimental.pallas import tpu_sc as plsc

_C = 32


@functools.partial(jax.jit, static_argnums=(2, 3, 4))
def _gather_call(input, weight, B, V, D):
    info = plsc.get_sparse_core_info()
    NC = info.num_cores
    L = info.num_lanes
    NW = NC * info.num_subcores
    t_w = B // NW
    n_chunk = t_w // _C
    table3 = weight.reshape(V // 8, 8, D)
    mesh = plsc.VectorSubcoreMesh(core_axis_name="c", subcore_axis_name="s")

    @functools.partial(
        pl.kernel,
        mesh=mesh,
        compiler_params=pltpu.CompilerParams(needs_layout_passes=False),
        out_type=jax.ShapeDtypeStruct((B // 8, 8, D), jnp.float32),
        scratch_types=[
            pltpu.VMEM((t_w,), jnp.int32),
            pltpu.VMEM((t_w,), jnp.int32),
            pltpu.VMEM((t_w,), jnp.int32),
            pltpu.VMEM((_C, 8, D), jnp.float32),
            pltpu.VMEM((_C // 8, 8, D), jnp.float32),
            pltpu.SemaphoreType.DMA,
        ],
    )
    def k(idx_hbm, table_hbm, out_hbm, idx_v, tidx_v, sub_v, gbuf, obuf, sem):
        wid = lax.axis_index("s") * NC + lax.axis_index("c")
        base = wid * t_w
        obase = wid * (t_w // 8)
        pltpu.sync_copy(idx_hbm.at[pl.ds(base, t_w)], idx_v)

        def split(j, carry):
            v = idx_v[pl.ds(j * L, L)]
            tidx_v[pl.ds(j * L, L)] = lax.shift_right_logical(v, 3)
            sub_v[pl.ds(j * L, L)] = lax.bitwise_and(v, 7)
            return carry

        lax.fori_loop(0, t_w // L, split, 0)

        def chunk(c, carry):
            off = c * _C
            copies = []
            for g in range(_C // L):
                tv = tidx_v[pl.ds(off + g * L, L)]
                for j in range(L):
                    copies.append(
                        pltpu.async_copy(
                            table_hbm.at[pl.ds(tv[j], 1)],
                            gbuf.at[pl.ds(g * L + j, 1)],
                            sem,
                        )
                    )
            for cp in copies:
                cp.wait()
            for g in range(_C // L):
                tloc = lax.iota(jnp.int32, L) + g * L
                trow = lax.shift_right_logical(tloc, 3)
                tsub = lax.bitwise_and(tloc, 7)
                svec = sub_v[pl.ds(off + g * L, L)]
                for col in range(0, D, L):
                    for col2 in range(L):
                        lvec = jnp.full((L,), col + col2, jnp.int32)
                        vals = plsc.load_gather(gbuf, [tloc, svec, lvec])
                        plsc.store_scatter(obuf, [trow, tsub, lvec], vals)
            pltpu.sync_copy(
                obuf, out_hbm.at[pl.ds(obase + c * (_C // 8), _C // 8)]
            )
            return carry

        lax.fori_loop(0, n_chunk, chunk, 0)

    out3 = k(input, table3)
    return out3.reshape(B, D)


def kernel(input, weight):
    (B,) = input.shape
    V, D = weight.shape
    return _gather_call(input, weight, B, V, D)
